# baseline (speedup 1.0000x reference)
import functools

import jax
import jax.numpy as jnp
from jax.experimental import pallas as pl
from jax.experimental.pallas import tpu as pltpu

_EPS = 1e-5


def _ceil_to(x, m):
    return ((x + m - 1) // m) * m


def _build_cols(pad_sc, col_sc, H, W, Cin, base):
    HW = H * W
    lane = jax.lax.broadcasted_iota(jnp.int32, (1, HW), 1)
    mask_l = (lane % W) != 0
    mask_r = (lane % W) != (W - 1)
    for dy in range(3):
        for dx in range(3):
            o = base + (dy - 1) * W + (dx - 1)
            tap = pad_sc[:, o:o + HW]
            if dx == 0:
                tap = jnp.where(mask_l, tap, jnp.bfloat16(0))
            elif dx == 2:
                tap = jnp.where(mask_r, tap, jnp.bfloat16(0))
            t = 3 * dy + dx
            col_sc[t * Cin:(t + 1) * Cin, :] = tap


def _stats8(acc):
    C = acc.shape[0]
    s1 = jnp.sum(acc, axis=1, keepdims=True)
    s2 = jnp.sum(acc * acc, axis=1, keepdims=True)
    return jnp.concatenate([s1, s2, jnp.zeros((C, 6), jnp.float32)], axis=1)


def _conv1_body(H, W, base, x_ref, w_ref, y_ref, st_ref, pad_sc, col_sc):
    Cin = x_ref.shape[1]
    HW = H * W
    P = pad_sc.shape[1]
    pad_sc[:, 0:base] = jnp.zeros((Cin, base), jnp.bfloat16)
    pad_sc[:, base + HW:P] = jnp.zeros((Cin, P - base - HW), jnp.bfloat16)
    pad_sc[:, base:base + HW] = x_ref[0].astype(jnp.bfloat16)
    _build_cols(pad_sc, col_sc, H, W, Cin, base)
    acc = jnp.dot(w_ref[...], col_sc[...],
                  preferred_element_type=jnp.float32)
    y_ref[0] = acc.astype(jnp.bfloat16)
    st_ref[0] = _stats8(acc)


def _conv2_body(H, W, base, y1_ref, ss_ref, w_ref, y_ref, st_ref,
                pad_sc, col_sc):
    Cmid = y1_ref.shape[1]
    HW = H * W
    P = pad_sc.shape[1]
    scale = ss_ref[:, 0:1]
    shift = ss_ref[:, 1:2]
    h = jnp.maximum(y1_ref[0].astype(jnp.float32) * scale + shift, 0.0)
    pad_sc[:, 0:base] = jnp.zeros((Cmid, base), jnp.bfloat16)
    pad_sc[:, base + HW:P] = jnp.zeros((Cmid, P - base - HW), jnp.bfloat16)
    pad_sc[:, base:base + HW] = h.astype(jnp.bfloat16)
    _build_cols(pad_sc, col_sc, H, W, Cmid, base)
    acc = jnp.dot(w_ref[...], col_sc[...],
                  preferred_element_type=jnp.float32)
    y_ref[0] = acc.astype(jnp.bfloat16)
    st_ref[0] = _stats8(acc)


def _bn_apply_body(y_ref, ss_ref, out_ref):
    scale = ss_ref[:, 0:1]
    shift = ss_ref[:, 1:2]
    out_ref[0] = y_ref[0].astype(jnp.float32) * scale + shift


def _scale_shift8(st, gamma, beta, count):
    s1 = jnp.sum(st[:, :, 0], axis=0)
    s2 = jnp.sum(st[:, :, 1], axis=0)
    mean = s1 / count
    var = jnp.maximum(s2 / count - mean * mean, 0.0)
    scale = gamma.reshape(-1) * jax.lax.rsqrt(var + _EPS)
    shift = beta.reshape(-1) - mean * scale
    C = scale.shape[0]
    return jnp.concatenate(
        [scale[:, None], shift[:, None], jnp.zeros((C, 6), jnp.float32)],
        axis=1)


def kernel(x, w1, b1, g1, be1, w2, b2, g2, be2):
    del b1, b2
    N, Cin, H, W = x.shape
    Cmid = w1.shape[-1]
    Cout = w2.shape[-1]
    HW = H * W
    base = _ceil_to(W + 1, max(2 * W, 8))
    P = _ceil_to(base + HW + W + 1, 128)
    count = float(N * HW)

    xf = x.reshape(N, Cin, HW)
    w1r = jnp.transpose(w1, (3, 0, 1, 2)).reshape(Cmid, 9 * Cin)
    w1r = w1r.astype(jnp.bfloat16)
    w2r = jnp.transpose(w2, (3, 0, 1, 2)).reshape(Cout, 9 * Cmid)
    w2r = w2r.astype(jnp.bfloat16)

    cp = pltpu.CompilerParams(
        dimension_semantics=("parallel",),
        vmem_limit_bytes=64 * 1024 * 1024,
    )

    ce1 = pl.CostEstimate(
        flops=2 * N * HW * 9 * Cin * Cmid, transcendentals=0,
        bytes_accessed=4 * N * HW * Cin + 2 * N * HW * Cmid)
    y1, st1 = pl.pallas_call(
        functools.partial(_conv1_body, H, W, base),
        grid=(N,),
        in_specs=[
            pl.BlockSpec((1, Cin, HW), lambda n: (n, 0, 0)),
            pl.BlockSpec((Cmid, 9 * Cin), lambda n: (0, 0)),
        ],
        out_specs=(
            pl.BlockSpec((1, Cmid, HW), lambda n: (n, 0, 0)),
            pl.BlockSpec((1, Cmid, 8), lambda n: (n, 0, 0)),
        ),
        out_shape=(
            jax.ShapeDtypeStruct((N, Cmid, HW), jnp.bfloat16),
            jax.ShapeDtypeStruct((N, Cmid, 8), jnp.float32),
        ),
        scratch_shapes=[
            pltpu.VMEM((Cin, P), jnp.bfloat16),
            pltpu.VMEM((9 * Cin, HW), jnp.bfloat16),
        ],
        compiler_params=cp,
        cost_estimate=ce1,
    )(xf, w1r)

    ss1 = _scale_shift8(st1, g1.astype(jnp.float32), be1.astype(jnp.float32),
                        count)

    ce2 = pl.CostEstimate(
        flops=2 * N * HW * 9 * Cmid * Cout, transcendentals=0,
        bytes_accessed=2 * N * HW * (Cmid + Cout))
    y2, st2 = pl.pallas_call(
        functools.partial(_conv2_body, H, W, base),
        grid=(N,),
        in_specs=[
            pl.BlockSpec((1, Cmid, HW), lambda n: (n, 0, 0)),
            pl.BlockSpec((Cmid, 8), lambda n: (0, 0)),
            pl.BlockSpec((Cout, 9 * Cmid), lambda n: (0, 0)),
        ],
        out_specs=(
            pl.BlockSpec((1, Cout, HW), lambda n: (n, 0, 0)),
            pl.BlockSpec((1, Cout, 8), lambda n: (n, 0, 0)),
        ),
        out_shape=(
            jax.ShapeDtypeStruct((N, Cout, HW), jnp.bfloat16),
            jax.ShapeDtypeStruct((N, Cout, 8), jnp.float32),
        ),
        scratch_shapes=[
            pltpu.VMEM((Cmid, P), jnp.bfloat16),
            pltpu.VMEM((9 * Cmid, HW), jnp.bfloat16),
        ],
        compiler_params=cp,
        cost_estimate=ce2,
    )(y1, ss1, w2r)

    ss2 = _scale_shift8(st2, g2.astype(jnp.float32), be2.astype(jnp.float32),
                        count)

    ce3 = pl.CostEstimate(
        flops=2 * N * HW * Cout, transcendentals=0,
        bytes_accessed=6 * N * HW * Cout)
    out = pl.pallas_call(
        _bn_apply_body,
        grid=(N,),
        in_specs=[
            pl.BlockSpec((1, Cout, HW), lambda n: (n, 0, 0)),
            pl.BlockSpec((Cout, 8), lambda n: (0, 0)),
        ],
        out_specs=pl.BlockSpec((1, Cout, HW), lambda n: (n, 0, 0)),
        out_shape=jax.ShapeDtypeStruct((N, Cout, HW), jnp.float32),
        compiler_params=cp,
        cost_estimate=ce3,
    )(y2, ss2)

    return out.reshape(N, Cout, H, W)
